# baseline (speedup 1.0000x reference)
import functools

import jax
import jax.numpy as jnp
from jax import lax
from jax.experimental import pallas as pl
from jax.experimental.pallas import tpu as pltpu
from jax.experimental.pallas import tpu_sc as plsc

B, CIN, W = 16, 128, 8192
COUT = 64
HALVES = 2
D = W // HALVES
NROWS_IN = B * CIN * HALVES
NROWS_OUT = B * COUT * HALVES

NC, NS, L = 2, 16, 16
NW = NC * NS
RPW = NROWS_OUT // NW
K = 8
NCHUNK = RPW // K


def _sc_body(x_hbm, rowidx_hbm, out_hbm, rowidx, buf0, buf1,
             gsem0, gsem1, psem0, psem1):
    wid = lax.axis_index("s") * NC + lax.axis_index("c")

    pltpu.sync_copy(rowidx_hbm.at[pl.ds(wid * RPW, RPW)], rowidx)

    bufs = (buf0, buf1)
    gsems = (gsem0, gsem1)
    psems = (psem0, psem1)
    out_base = wid * RPW

    def g_copy(ch, X):
        return pltpu.make_async_copy(
            x_hbm.at[rowidx.at[pl.ds(ch * K, K)]], bufs[X], gsems[X])

    def p_copy(ch, X):
        return pltpu.make_async_copy(
            bufs[X], out_hbm.at[pl.ds(out_base + ch * K, K)], psems[X])

    g_copy(0, 0).start()
    g_copy(1, 1).start()
    for ch in range(NCHUNK):
        X = ch % 2
        g_copy(ch, X).wait()
        p_copy(ch, X).start()
        if ch + 2 < NCHUNK:
            p_copy(ch, X).wait()
            g_copy(ch + 2, X).start()
    p_copy(NCHUNK - 2, (NCHUNK - 2) % 2).wait()
    p_copy(NCHUNK - 1, (NCHUNK - 1) % 2).wait()


@jax.jit
def _sc_gather(x2, rowidx_all):
    mesh = plsc.VectorSubcoreMesh(core_axis_name="c", subcore_axis_name="s")
    f = functools.partial(
        pl.kernel,
        mesh=mesh,
        out_type=jax.ShapeDtypeStruct((NROWS_OUT, D), jnp.float32),
        scratch_types=[
            pltpu.VMEM((RPW,), jnp.int32),
            pltpu.VMEM((K, D), jnp.float32),
            pltpu.VMEM((K, D), jnp.float32),
            pltpu.SemaphoreType.DMA,
            pltpu.SemaphoreType.DMA,
            pltpu.SemaphoreType.DMA,
            pltpu.SemaphoreType.DMA,
        ],
    )(_sc_body)
    return f(x2, rowidx_all)


def kernel(x, idx):
    x2 = x.reshape(NROWS_IN, D)
    gid = jnp.arange(B, dtype=jnp.int32)[:, None] * CIN + idx[None, :]
    rowidx_all = (gid[:, :, None] * HALVES
                  + jnp.arange(HALVES, dtype=jnp.int32)).reshape(-1)
    out2 = _sc_gather(x2, rowidx_all)
    return out2.reshape(B, COUT, W)


# speedup vs baseline: 3.3591x; 3.3591x over previous
import jax
import jax.numpy as jnp
from jax import lax
from jax.experimental import pallas as pl
from jax.experimental.pallas import tpu as pltpu
from jax.experimental.pallas import tpu_sc as plsc
from jax._src.pallas import mpmd

B, CIN, W = 16, 128, 8192
COUT = 64
NROWS_IN = B * CIN
NROWS_OUT = B * COUT

NC, NS = 2, 16
NW = NC * NS

NT = 640
RPW = NT // NW
K = 2
NCHUNK = RPW // K
NBUF = 6

RPS = (NROWS_OUT - NT) // NC
RPSP = 256
NB = 8
NCH_S = RPS // NB
NBANK = 4


def _tec_body(x_hbm, ridx_tec, ridx_scs, out_hbm,
              rowidx, tbufs, tgsems, tpsems,
              spbuf, sgsems, spsems):
    del ridx_scs, spbuf, sgsems, spsems

    wid = lax.axis_index("s") * NC + lax.axis_index("c")
    pltpu.sync_copy(ridx_tec.at[wid], rowidx)
    out_base = wid * RPW

    def g_copy(ch, X):
        return pltpu.make_async_copy(
            x_hbm.at[rowidx.at[ch]], tbufs[X], tgsems[X])

    def p_copy(ch, X):
        return pltpu.make_async_copy(
            tbufs[X], out_hbm.at[pl.ds(out_base + ch * K, K)], tpsems[X])

    for i in range(min(NBUF - 1, NCHUNK)):
        g_copy(i, i).start()
    for ch in range(NCHUNK):
        X = ch % NBUF
        g_copy(ch, X).wait()
        p_copy(ch, X).start()
        nxt = ch + NBUF - 1
        if nxt < NCHUNK:
            if nxt >= NBUF:
                p_copy(nxt - NBUF, nxt % NBUF).wait()
            g_copy(nxt, nxt % NBUF).start()
    for ch in range(max(0, NCHUNK - NBUF), NCHUNK):
        p_copy(ch, ch % NBUF).wait()


def _scs_body(x_hbm, ridx_tec, ridx_scs, out_hbm,
              rowidx, tbufs, tgsems, tpsems,
              spbuf, sgsems, spsems):
    del ridx_tec, rowidx, tbufs, tgsems, tpsems

    def inner(idx_smem):
        _scs_inner(x_hbm, ridx_scs, out_hbm, idx_smem, spbuf, sgsems, spsems)

    pl.run_scoped(inner, pltpu.SMEM((RPSP,), jnp.int32))


def _scs_inner(x_hbm, ridx_scs, out_hbm, idx_smem, spbuf, sgsems, spsems):
    c = lax.axis_index("c")
    pltpu.sync_copy(ridx_scs.at[pl.ds(c * RPSP, RPSP)], idx_smem)
    out_base = NT + c * RPS

    def g_descs(ch, X):
        return [pltpu.make_async_copy(
                    x_hbm.at[pl.ds(idx_smem[ch * NB + j], 1)],
                    spbuf.at[pl.ds(X * NB + j, 1)],
                    sgsems[X])
                for j in range(NB)]

    def p_copy(ch, X):
        return pltpu.make_async_copy(
            spbuf.at[pl.ds(X * NB, NB)],
            out_hbm.at[pl.ds(out_base + ch * NB, NB)], spsems[X])

    for i in range(min(NBANK - 1, NCH_S)):
        for d in g_descs(i, i):
            d.start()
    for ch in range(NCH_S):
        X = ch % NBANK
        for d in g_descs(ch, X):
            d.wait()
        p_copy(ch, X).start()
        nxt = ch + NBANK - 1
        if nxt < NCH_S:
            if nxt >= NBANK:
                p_copy(nxt - NBANK, nxt % NBANK).wait()
            for d in g_descs(nxt, nxt % NBANK):
                d.start()
    for ch in range(max(0, NCH_S - NBANK), NCH_S):
        p_copy(ch, ch % NBANK).wait()


@jax.jit
def _sc_gather(x2, ridx_tec, ridx_scs):
    vmesh = plsc.VectorSubcoreMesh(core_axis_name="c", subcore_axis_name="s")
    smesh = plsc.ScalarSubcoreMesh(axis_name="c", num_cores=NC)
    f = mpmd.mpmd_map(
        [(smesh, _scs_body), (vmesh, _tec_body)],
        out_types=[jax.ShapeDtypeStruct((NROWS_OUT, W), jnp.float32)],
        scratch_types=(
            (pltpu.VMEM @ vmesh)((NCHUNK, K), jnp.int32),
            tuple((pltpu.VMEM @ vmesh)((K, W), jnp.float32)
                  for _ in range(NBUF)),
            tuple(pltpu.SemaphoreType.DMA @ vmesh for _ in range(NBUF)),
            tuple(pltpu.SemaphoreType.DMA @ vmesh for _ in range(NBUF)),
            pltpu.VMEM_SHARED((NBANK * NB, W), jnp.float32),
            tuple(pltpu.SemaphoreType.DMA @ smesh for _ in range(NBANK)),
            tuple(pltpu.SemaphoreType.DMA @ smesh for _ in range(NBANK)),
        ),
    )
    return f(x2, ridx_tec, ridx_scs)


def kernel(x, idx):
    x2 = x.reshape(NROWS_IN, W)
    gid = (jnp.arange(B, dtype=jnp.int32)[:, None] * CIN
           + idx[None, :]).reshape(NROWS_OUT)
    ridx_tec = gid[:NT].reshape(NW, NCHUNK, K)
    ridx_scs = jnp.pad(gid[NT:].reshape(NC, RPS),
                       ((0, 0), (0, RPSP - RPS))).reshape(-1)
    (out2,) = _sc_gather(x2, ridx_tec, ridx_scs)
    return out2.reshape(B, COUT, W)
